# baseline (speedup 1.0000x reference)
import jax
import jax.numpy as jnp
from jax.experimental import pallas as pl
from jax.experimental.pallas import tpu as pltpu

TM = 256
NB = 512
KPAD = 256


def _moe_body(xb_ref, wr_ref, br_ref, w_ref, out_ref, xg_ref):
    E = wr_ref.shape[1]
    D = xb_ref.shape[1]

    xb = xb_ref[...]
    logits = jnp.dot(
        xb, wr_ref[...], preferred_element_type=jnp.float32
    ) + br_ref[...]
    m = jnp.max(logits, axis=1, keepdims=True)
    p = jnp.exp(logits - m)
    gate = p / jnp.sum(p, axis=1, keepdims=True)
    gate_bf = gate.astype(jnp.bfloat16)
    for e in range(E):
        xg_ref[:, e * D:(e + 1) * D] = xb * gate_bf[:, e:e + 1]
    tail = jnp.concatenate(
        [gate_bf, jnp.zeros((TM, KPAD - E), jnp.bfloat16)], axis=1
    )
    xg_ref[:, E * D:] = tail

    out_ref[...] = jnp.dot(
        xg_ref[...], w_ref[...], preferred_element_type=jnp.float32
    )


@jax.jit
def kernel(x, Wr, br, We, be):
    T, D = x.shape
    E, _, _ = We.shape
    K = E * D + KPAD
    nt = T // TM
    nn = D // NB

    xb = x.astype(jnp.bfloat16)
    wr_bf = Wr.astype(jnp.bfloat16)
    br2 = br.reshape(1, E)
    w_full = jnp.concatenate(
        [We.reshape(E * D, D), be, jnp.zeros((KPAD - E, D), We.dtype)], axis=0
    ).astype(jnp.bfloat16)

    return pl.pallas_call(
        _moe_body,
        grid=(nn, nt),
        in_specs=[
            pl.BlockSpec((TM, D), lambda n, t: (t, 0)),
            pl.BlockSpec((D, E), lambda n, t: (0, 0)),
            pl.BlockSpec((1, E), lambda n, t: (0, 0)),
            pl.BlockSpec((K, NB), lambda n, t: (0, n)),
        ],
        out_specs=pl.BlockSpec((TM, NB), lambda n, t: (t, n)),
        out_shape=jax.ShapeDtypeStruct((T, D), jnp.float32),
        scratch_shapes=[
            pltpu.VMEM((TM, K), jnp.bfloat16),
        ],
        compiler_params=pltpu.CompilerParams(
            dimension_semantics=("arbitrary", "arbitrary"),
        ),
    )(xb, wr_bf, br2, w_full)


# speedup vs baseline: 1.1257x; 1.1257x over previous
import jax
import jax.numpy as jnp
from jax.experimental import pallas as pl
from jax.experimental.pallas import tpu as pltpu

TM = 1024
NB = 256
KPAD = 128


def _moe_body(xb_ref, wr_ref, br_ref, w_ref, out_ref, xg_ref):
    n = pl.program_id(1)
    E = wr_ref.shape[1]
    D = xb_ref.shape[1]

    @pl.when(n == 0)
    def _build():
        xb = xb_ref[...]
        logits = jnp.dot(
        xb, wr_ref[...], preferred_element_type=jnp.float32
        ) + br_ref[...]
        m = jnp.max(logits, axis=1, keepdims=True)
        p = jnp.exp(logits - m)
        gate = p / jnp.sum(p, axis=1, keepdims=True)
        gate_bf = gate.astype(jnp.bfloat16)
        for e in range(E):
            xg_ref[:, e * D:(e + 1) * D] = xb * gate_bf[:, e:e + 1]
        tail = jnp.concatenate(
            [gate_bf, jnp.zeros((TM, KPAD - E), jnp.bfloat16)], axis=1
        )
        xg_ref[:, E * D:] = tail

    out_ref[...] = jnp.dot(
        xg_ref[...], w_ref[...], preferred_element_type=jnp.float32
    )


@jax.jit
def kernel(x, Wr, br, We, be):
    T, D = x.shape
    E, _, _ = We.shape
    K = E * D + KPAD
    nt = T // TM
    nn = D // NB

    xb = x.astype(jnp.bfloat16)
    wr_bf = Wr.astype(jnp.bfloat16)
    br2 = br.reshape(1, E)
    w_full = jnp.concatenate(
        [We.reshape(E * D, D), be, jnp.zeros((KPAD - E, D), We.dtype)], axis=0
    ).astype(jnp.bfloat16)

    return pl.pallas_call(
        _moe_body,
        grid=(nt, nn),
        in_specs=[
            pl.BlockSpec((TM, D), lambda t, n: (t, 0)),
            pl.BlockSpec((D, E), lambda t, n: (0, 0)),
            pl.BlockSpec((1, E), lambda t, n: (0, 0)),
            pl.BlockSpec((K, NB), lambda t, n: (0, n)),
        ],
        out_specs=pl.BlockSpec((TM, NB), lambda t, n: (t, n)),
        out_shape=jax.ShapeDtypeStruct((T, D), jnp.float32),
        scratch_shapes=[
            pltpu.VMEM((TM, K), jnp.bfloat16),
        ],
        compiler_params=pltpu.CompilerParams(
            dimension_semantics=("arbitrary", "arbitrary"),
            vmem_limit_bytes=64 * 1024 * 1024,
        ),
    )(xb, wr_bf, br2, w_full)


# speedup vs baseline: 1.2030x; 1.0686x over previous
import jax
import jax.numpy as jnp
import numpy as np
from jax.experimental import pallas as pl
from jax.experimental.pallas import tpu as pltpu
from jax.experimental.shard_map import shard_map
from jax.sharding import Mesh, PartitionSpec as P

TM = 1024
NB = 256
KPAD = 128


def _moe_body(xb_ref, wr_ref, br_ref, w_ref, out_ref, xg_ref):
    n = pl.program_id(1)
    E = wr_ref.shape[1]
    D = xb_ref.shape[1]

    @pl.when(n == 0)
    def _build():
        xb = xb_ref[...]
        logits = jnp.dot(
        xb, wr_ref[...], preferred_element_type=jnp.float32
        ) + br_ref[...]
        m = jnp.max(logits, axis=1, keepdims=True)
        p = jnp.exp(logits - m)
        gate = p / jnp.sum(p, axis=1, keepdims=True)
        gate_bf = gate.astype(jnp.bfloat16)
        for e in range(E):
            xg_ref[:, e * D:(e + 1) * D] = xb * gate_bf[:, e:e + 1]
        tail = jnp.concatenate(
            [gate_bf, jnp.zeros((TM, KPAD - E), jnp.bfloat16)], axis=1
        )
        xg_ref[:, E * D:] = tail

    out_ref[...] = jnp.dot(
        xg_ref[...], w_ref[...], preferred_element_type=jnp.float32
    )


def _moe_call(xb, wr_bf, br2, w_full):
    Tl, D = xb.shape
    K = w_full.shape[0]
    return pl.pallas_call(
        _moe_body,
        grid=(Tl // TM, D // NB),
        in_specs=[
            pl.BlockSpec((TM, D), lambda t, n: (t, 0)),
            pl.BlockSpec((D, wr_bf.shape[1]), lambda t, n: (0, 0)),
            pl.BlockSpec((1, br2.shape[1]), lambda t, n: (0, 0)),
            pl.BlockSpec((K, NB), lambda t, n: (0, n)),
        ],
        out_specs=pl.BlockSpec((TM, NB), lambda t, n: (t, n)),
        out_shape=jax.ShapeDtypeStruct((Tl, D), jnp.float32),
        scratch_shapes=[
            pltpu.VMEM((TM, K), jnp.bfloat16),
        ],
        compiler_params=pltpu.CompilerParams(
            dimension_semantics=("arbitrary", "arbitrary"),
            vmem_limit_bytes=64 * 1024 * 1024,
        ),
    )(xb, wr_bf, br2, w_full)


@jax.jit
def kernel(x, Wr, br, We, be):
    T, D = x.shape
    E, _, _ = We.shape

    xb = x.astype(jnp.bfloat16)
    wr_bf = Wr.astype(jnp.bfloat16)
    br2 = br.reshape(1, E)
    w_full = jnp.concatenate(
        [We.reshape(E * D, D), be, jnp.zeros((KPAD - E, D), We.dtype)], axis=0
    ).astype(jnp.bfloat16)

    devs = jax.devices()
    ndev = 1
    for cand in (4, 2):
        if len(devs) >= cand and (T // TM) % cand == 0:
            ndev = cand
            break
    if ndev > 1:
        mesh = Mesh(np.array(devs[:ndev]), ("d",))
        f = shard_map(
            _moe_call,
            mesh=mesh,
            in_specs=(P("d", None), P(None, None), P(None, None), P(None, None)),
            out_specs=P("d", None),
            check_rep=False,
        )
        return f(xb, wr_bf, br2, w_full)
    return _moe_call(xb, wr_bf, br2, w_full)


# speedup vs baseline: 1.2191x; 1.0133x over previous
import jax
import jax.numpy as jnp
import numpy as np
from jax.experimental import pallas as pl
from jax.experimental.pallas import tpu as pltpu
from jax.experimental.shard_map import shard_map
from jax.sharding import Mesh, PartitionSpec as P

TM = 1024
NB = 256
KPAD = 128


def _moe_body(xb_ref, wr_ref, br_ref, w_ref, out_ref, xg_ref):
    n = pl.program_id(1)
    E = wr_ref.shape[1]
    D = xb_ref.shape[1]

    @pl.when(n == 0)
    def _build():
        xb = xb_ref[...]
        logits = jnp.dot(
        xb, wr_ref[...], preferred_element_type=jnp.float32
        ) + br_ref[...]
        m = jnp.max(logits, axis=1, keepdims=True)
        p = jnp.exp(logits - m)
        gate = p / jnp.sum(p, axis=1, keepdims=True)
        gate_bf = gate.astype(jnp.bfloat16)
        for e in range(E):
            xg_ref[:, e * D:(e + 1) * D] = xb * gate_bf[:, e:e + 1]
        tail = jnp.concatenate(
            [gate_bf, jnp.zeros((TM, KPAD - E), jnp.bfloat16)], axis=1
        )
        xg_ref[:, E * D:] = tail

    out_ref[...] = jnp.dot(
        xg_ref[...], w_ref[...], preferred_element_type=jnp.float32
    )


def _moe_call(xb, wr_bf, br2, w_full):
    Tl, D = xb.shape
    K = w_full.shape[0]
    return pl.pallas_call(
        _moe_body,
        grid=(Tl // TM, D // NB),
        in_specs=[
            pl.BlockSpec((TM, D), lambda t, n: (t, 0)),
            pl.BlockSpec((D, wr_bf.shape[1]), lambda t, n: (0, 0)),
            pl.BlockSpec((1, br2.shape[1]), lambda t, n: (0, 0)),
            pl.BlockSpec((K, NB), lambda t, n: (0, n)),
        ],
        out_specs=pl.BlockSpec((TM, NB), lambda t, n: (t, n)),
        out_shape=jax.ShapeDtypeStruct((Tl, D), jnp.float32),
        scratch_shapes=[
            pltpu.VMEM((TM, K), jnp.bfloat16),
        ],
        compiler_params=pltpu.CompilerParams(
            dimension_semantics=("arbitrary", "arbitrary"),
            vmem_limit_bytes=64 * 1024 * 1024,
        ),
    )(xb, wr_bf, br2, w_full)


@jax.jit
def kernel(x, Wr, br, We, be):
    T, D = x.shape
    E, _, _ = We.shape

    _devs = jax.devices()
    _nd = 2 if len(_devs) >= 2 and (T // TM) % 2 == 0 else 1
    if _nd > 1:
        _mesh = Mesh(np.array(_devs[:_nd]), ("d",))
        _sh = lambda a, spec: jax.lax.with_sharding_constraint(
            a, jax.sharding.NamedSharding(_mesh, spec))
        x = _sh(x, P("d", None))
        Wr = _sh(Wr, P(None, None))
        br = _sh(br, P(None))
        We = _sh(We, P(None, None, None))
        be = _sh(be, P(None, None))

    xb = x.astype(jnp.bfloat16)
    wr_bf = Wr.astype(jnp.bfloat16)
    br2 = br.reshape(1, E)
    w_full = jnp.concatenate(
        [We.reshape(E * D, D), be, jnp.zeros((KPAD - E, D), We.dtype)], axis=0
    ).astype(jnp.bfloat16)

    devs = jax.devices()
    ndev = 1
    for cand in (4, 2):
        if len(devs) >= cand and (T // TM) % cand == 0:
            ndev = cand
            break
    if ndev > 1:
        mesh = Mesh(np.array(devs[:ndev]), ("d",))
        f = shard_map(
            _moe_call,
            mesh=mesh,
            in_specs=(P("d", None), P(None, None), P(None, None), P(None, None)),
            out_specs=P("d", None),
            check_rep=False,
        )
        return f(xb, wr_bf, br2, w_full)
    return _moe_call(xb, wr_bf, br2, w_full)
